# baseline (speedup 1.0000x reference)
import jax
import jax.numpy as jnp
from jax import lax
from jax.experimental import pallas as pl
from jax.experimental.pallas import tpu as pltpu
from jax.experimental.pallas import tpu_sc as plsc

VOCAB = 1000000
EMBED_DIM = 64
BATCH = 16384

NUM_CORES = 2
NUM_SUBCORES = 16
LANES = 16

NUM_WORKERS = NUM_CORES * NUM_SUBCORES
ROWS_PER_WORKER = BATCH // NUM_WORKERS
GATHER_CHUNK = 128
NUM_GATHERS = ROWS_PER_WORKER // GATHER_CHUNK


def _sc_body(cw_hbm, xw_hbm, ctab_hbm, xtab_hbm, out_hbm,
             idxc_v, idxx_v, rowsc_v, rowsx_v, out_v, sems):
    wid = lax.axis_index("s") * NUM_CORES + lax.axis_index("c")
    base = wid * ROWS_PER_WORKER

    pltpu.sync_copy(cw_hbm.at[pl.ds(base, ROWS_PER_WORKER)], idxc_v)
    pltpu.sync_copy(xw_hbm.at[pl.ds(base, ROWS_PER_WORKER)], idxx_v)

    copies = []
    for g in range(NUM_GATHERS):
        sl = pl.ds(g * GATHER_CHUNK, GATHER_CHUNK)
        copies.append(pltpu.async_copy(
            ctab_hbm.at[idxc_v.at[sl]], rowsc_v.at[sl], sems.at[0]))
        copies.append(pltpu.async_copy(
            xtab_hbm.at[idxx_v.at[sl]], rowsx_v.at[sl], sems.at[1]))
    for c in copies:
        c.wait()

    lane_iota = lax.iota(jnp.int32, LANES)

    def chunk_body(chunk, carry):
        row_ids = chunk * LANES + lane_iota
        acc = jnp.zeros((LANES,), jnp.float32)
        for d in range(EMBED_DIM):
            col = jnp.full((LANES,), d, jnp.int32)
            c = plsc.load_gather(rowsc_v, [row_ids, col])
            x = plsc.load_gather(rowsx_v, [row_ids, col])
            acc = acc + c * x
        sig = 1.0 / (1.0 + jnp.exp(-acc))
        out_v[pl.ds(chunk * LANES, LANES)] = sig
        return carry

    lax.fori_loop(0, ROWS_PER_WORKER // LANES, chunk_body, 0)

    pltpu.sync_copy(out_v, out_hbm.at[pl.ds(base, ROWS_PER_WORKER)])


def kernel(center_words, context_words, center_table, context_table):
    mesh = plsc.VectorSubcoreMesh(core_axis_name="c", subcore_axis_name="s")
    run = pl.kernel(
        _sc_body,
        out_type=jax.ShapeDtypeStruct((BATCH,), jnp.float32),
        mesh=mesh,
        scratch_types=[
            pltpu.VMEM((ROWS_PER_WORKER,), jnp.int32),
            pltpu.VMEM((ROWS_PER_WORKER,), jnp.int32),
            pltpu.VMEM((ROWS_PER_WORKER, EMBED_DIM), jnp.float32),
            pltpu.VMEM((ROWS_PER_WORKER, EMBED_DIM), jnp.float32),
            pltpu.VMEM((ROWS_PER_WORKER,), jnp.float32),
            pltpu.SemaphoreType.DMA((2,)),
        ],
        compiler_params=pltpu.CompilerParams(
            needs_layout_passes=False, use_tc_tiling_on_sc=False),
    )
    return run(center_words.astype(jnp.int32), context_words.astype(jnp.int32),
               center_table, context_table)


# speedup vs baseline: 1.5562x; 1.5562x over previous
import jax
import jax.numpy as jnp
from jax import lax
from jax.experimental import pallas as pl
from jax.experimental.pallas import tpu as pltpu
from jax.experimental.pallas import tpu_sc as plsc

VOCAB = 1000000
EMBED_DIM = 64
BATCH = 16384

NUM_CORES = 2
NUM_SUBCORES = 16
LANES = 16

NUM_WORKERS = NUM_CORES * NUM_SUBCORES
ROWS_PER_WORKER = BATCH // NUM_WORKERS
HALF = ROWS_PER_WORKER // 2


def _sc_body(cw_hbm, xw_hbm, ctab_hbm, xtab_hbm, out_hbm,
             idxc_v, idxx_v, rowsc_v, rowsx_v, out_v, sems):
    wid = lax.axis_index("s") * NUM_CORES + lax.axis_index("c")
    base = wid * ROWS_PER_WORKER

    pltpu.sync_copy(cw_hbm.at[pl.ds(base, ROWS_PER_WORKER)], idxc_v)
    pltpu.sync_copy(xw_hbm.at[pl.ds(base, ROWS_PER_WORKER)], idxx_v)

    lane_iota = lax.iota(jnp.int32, LANES)

    for p in range(2):
        def issue(g, carry):
            vc = idxc_v[pl.ds(p * HALF + g * LANES, LANES)]
            vx = idxx_v[pl.ds(p * HALF + g * LANES, LANES)]
            for k in range(LANES):
                pltpu.async_copy(ctab_hbm.at[vc[k]],
                                 rowsc_v.at[g * LANES + k], sems.at[0])
                pltpu.async_copy(xtab_hbm.at[vx[k]],
                                 rowsx_v.at[g * LANES + k], sems.at[1])
            return carry

        lax.fori_loop(0, HALF // LANES, issue, 0)

        def drain(j, carry):
            pltpu.make_async_copy(ctab_hbm.at[0], rowsc_v.at[j],
                                  sems.at[0]).wait()
            pltpu.make_async_copy(xtab_hbm.at[0], rowsx_v.at[j],
                                  sems.at[1]).wait()
            return carry

        lax.fori_loop(0, HALF, drain, 0)

        def chunk_body(chunk, carry):
            row_ids = chunk * LANES + lane_iota
            acc = jnp.zeros((LANES,), jnp.float32)
            for d in range(EMBED_DIM):
                col = jnp.full((LANES,), d, jnp.int32)
                c = plsc.load_gather(rowsc_v, [row_ids, col])
                x = plsc.load_gather(rowsx_v, [row_ids, col])
                acc = acc + c * x
            sig = 1.0 / (1.0 + jnp.exp(-acc))
            out_v[pl.ds(p * HALF + chunk * LANES, LANES)] = sig
            return carry

        lax.fori_loop(0, HALF // LANES, chunk_body, 0)

    pltpu.sync_copy(out_v, out_hbm.at[pl.ds(base, ROWS_PER_WORKER)])


def kernel(center_words, context_words, center_table, context_table):
    mesh = plsc.VectorSubcoreMesh(core_axis_name="c", subcore_axis_name="s")
    run = pl.kernel(
        _sc_body,
        out_type=jax.ShapeDtypeStruct((BATCH,), jnp.float32),
        mesh=mesh,
        scratch_types=[
            pltpu.VMEM((ROWS_PER_WORKER,), jnp.int32),
            pltpu.VMEM((ROWS_PER_WORKER,), jnp.int32),
            pltpu.VMEM((HALF, EMBED_DIM), jnp.float32),
            pltpu.VMEM((HALF, EMBED_DIM), jnp.float32),
            pltpu.VMEM((ROWS_PER_WORKER,), jnp.float32),
            pltpu.SemaphoreType.DMA((2,)),
        ],
        compiler_params=pltpu.CompilerParams(
            needs_layout_passes=False, use_tc_tiling_on_sc=True),
    )
    return run(center_words.astype(jnp.int32), context_words.astype(jnp.int32),
               center_table, context_table)


# speedup vs baseline: 2.8918x; 1.8583x over previous
import jax
import jax.numpy as jnp
from jax import lax
from jax.experimental import pallas as pl
from jax.experimental.pallas import tpu as pltpu
from jax.experimental.pallas import tpu_sc as plsc

VOCAB = 1000000
EMBED_DIM = 64
BATCH = 16384

NUM_CORES = 2
NUM_SUBCORES = 16
LANES = 16

NUM_WORKERS = NUM_CORES * NUM_SUBCORES
ROWS_PER_WORKER = BATCH // NUM_WORKERS
NUM_SLOTS = 4
IDX_PAD = ROWS_PER_WORKER + LANES


BLOCK = 128


def _fetch_row(tab_hbm, blocks_v, idx_v, j, sem):
    vq = idx_v[pl.ds(j, LANES)]
    v = vq[0]
    off = pl.multiple_of((v // BLOCK) * BLOCK, BLOCK)
    slot_off = pl.multiple_of((j % NUM_SLOTS) * BLOCK, BLOCK)
    pltpu.async_copy(tab_hbm.at[:, pl.ds(off, BLOCK)],
                     blocks_v.at[:, pl.ds(slot_off, BLOCK)], sem)


def _sc_body(cw_hbm, xw_hbm, ctab_hbm, xtab_hbm, out_hbm,
             idxc_v, idxx_v, blkc_v, blkx_v, out_v, sems):
    wid = lax.axis_index("s") * NUM_CORES + lax.axis_index("c")
    base = wid * ROWS_PER_WORKER

    pltpu.sync_copy(cw_hbm.at[pl.ds(base, ROWS_PER_WORKER)],
                    idxc_v.at[pl.ds(0, ROWS_PER_WORKER)])
    pltpu.sync_copy(xw_hbm.at[pl.ds(base, ROWS_PER_WORKER)],
                    idxx_v.at[pl.ds(0, ROWS_PER_WORKER)])

    lane_iota = lax.iota(jnp.int32, LANES)
    last_mask = lane_iota == (LANES - 1)

    def prime(j, carry):
        _fetch_row(ctab_hbm, blkc_v, idxc_v, j, sems.at[0])
        _fetch_row(xtab_hbm, blkx_v, idxx_v, j, sems.at[1])
        return carry

    lax.fori_loop(0, NUM_SLOTS, prime, 0)

    def step(j, carry):
        slot_off = pl.multiple_of((j % NUM_SLOTS) * BLOCK, BLOCK)
        vc = idxc_v[pl.ds(j, LANES)]
        vx = idxx_v[pl.ds(j, LANES)]
        rc = vc[0] % BLOCK
        rx = vx[0] % BLOCK

        pltpu.make_async_copy(ctab_hbm.at[:, pl.ds(0, BLOCK)],
                              blkc_v.at[:, pl.ds(slot_off, BLOCK)],
                              sems.at[0]).wait()
        pltpu.make_async_copy(xtab_hbm.at[:, pl.ds(0, BLOCK)],
                              blkx_v.at[:, pl.ds(slot_off, BLOCK)],
                              sems.at[1]).wait()

        colc = jnp.full((LANES,), slot_off, jnp.int32) + rc
        colx = jnp.full((LANES,), slot_off, jnp.int32) + rx
        acc = jnp.zeros((LANES,), jnp.float32)
        for k in range(EMBED_DIM // LANES):
            rows = lane_iota + k * LANES
            c = plsc.load_gather(blkc_v, [rows, colc])
            x = plsc.load_gather(blkx_v, [rows, colx])
            acc = acc + c * x
        total = plsc.cumsum(acc)
        plsc.store_scatter(out_v, [jnp.full((LANES,), j, jnp.int32)], total,
                           mask=last_mask)

        @pl.when(j + NUM_SLOTS < ROWS_PER_WORKER)
        def _():
            _fetch_row(ctab_hbm, blkc_v, idxc_v, j + NUM_SLOTS, sems.at[0])
            _fetch_row(xtab_hbm, blkx_v, idxx_v, j + NUM_SLOTS, sems.at[1])

        return carry

    lax.fori_loop(0, ROWS_PER_WORKER, step, 0)

    def sig_body(chunk, carry):
        sl = pl.ds(chunk * LANES, LANES)
        out_v[sl] = 1.0 / (1.0 + jnp.exp(-out_v[sl]))
        return carry

    lax.fori_loop(0, ROWS_PER_WORKER // LANES, sig_body, 0)

    pltpu.sync_copy(out_v, out_hbm.at[pl.ds(base, ROWS_PER_WORKER)])


def kernel(center_words, context_words, center_table, context_table):
    mesh = plsc.VectorSubcoreMesh(core_axis_name="c", subcore_axis_name="s")
    run = pl.kernel(
        _sc_body,
        out_type=jax.ShapeDtypeStruct((BATCH,), jnp.float32),
        mesh=mesh,
        scratch_types=[
            pltpu.VMEM((IDX_PAD,), jnp.int32),
            pltpu.VMEM((IDX_PAD,), jnp.int32),
            pltpu.VMEM((EMBED_DIM, NUM_SLOTS * BLOCK), jnp.float32),
            pltpu.VMEM((EMBED_DIM, NUM_SLOTS * BLOCK), jnp.float32),
            pltpu.VMEM((ROWS_PER_WORKER,), jnp.float32),
            pltpu.SemaphoreType.DMA((2,)),
        ],
        compiler_params=pltpu.CompilerParams(
            needs_layout_passes=False, use_tc_tiling_on_sc=True),
    )
    return run(center_words.astype(jnp.int32), context_words.astype(jnp.int32),
               center_table.T, context_table.T)


# speedup vs baseline: 3.0282x; 1.0472x over previous
import jax
import jax.numpy as jnp
from jax import lax
from jax.experimental import pallas as pl
from jax.experimental.pallas import tpu as pltpu
from jax.experimental.pallas import tpu_sc as plsc

VOCAB = 1000000
EMBED_DIM = 64
BATCH = 16384

NUM_CORES = 2
NUM_SUBCORES = 16
LANES = 16

NUM_WORKERS = NUM_CORES * NUM_SUBCORES
ROWS_PER_WORKER = BATCH // NUM_WORKERS
NUM_SLOTS = 6
IDX_PAD = ROWS_PER_WORKER + LANES


BLOCK = 128


def _fetch_row(tab_hbm, blocks_v, idx_v, j, sems, table_id):
    vq = idx_v[pl.ds(j, LANES)]
    v = vq[0]
    slot = j % NUM_SLOTS
    off = pl.multiple_of((v // BLOCK) * BLOCK, BLOCK)
    slot_off = pl.multiple_of(slot * BLOCK, BLOCK)
    pltpu.async_copy(tab_hbm.at[:, pl.ds(off, BLOCK)],
                     blocks_v.at[:, pl.ds(slot_off, BLOCK)],
                     sems.at[table_id * NUM_SLOTS + slot])


def _sc_body(cw_hbm, xw_hbm, ctab_hbm, xtab_hbm, out_hbm,
             idxc_v, idxx_v, blkc_v, blkx_v, out_v, sems):
    wid = lax.axis_index("s") * NUM_CORES + lax.axis_index("c")
    base = wid * ROWS_PER_WORKER

    pltpu.sync_copy(cw_hbm.at[pl.ds(base, ROWS_PER_WORKER)],
                    idxc_v.at[pl.ds(0, ROWS_PER_WORKER)])
    pltpu.sync_copy(xw_hbm.at[pl.ds(base, ROWS_PER_WORKER)],
                    idxx_v.at[pl.ds(0, ROWS_PER_WORKER)])

    lane_iota = lax.iota(jnp.int32, LANES)
    last_mask = lane_iota == (LANES - 1)

    def prime(j, carry):
        _fetch_row(ctab_hbm, blkc_v, idxc_v, j, sems, 0)
        _fetch_row(xtab_hbm, blkx_v, idxx_v, j, sems, 1)
        return carry

    lax.fori_loop(0, NUM_SLOTS, prime, 0)

    def step(j, carry):
        slot = j % NUM_SLOTS
        slot_off = pl.multiple_of(slot * BLOCK, BLOCK)
        vc = idxc_v[pl.ds(j, LANES)]
        vx = idxx_v[pl.ds(j, LANES)]
        rc = vc[0] % BLOCK
        rx = vx[0] % BLOCK

        pltpu.make_async_copy(ctab_hbm.at[:, pl.ds(0, BLOCK)],
                              blkc_v.at[:, pl.ds(slot_off, BLOCK)],
                              sems.at[slot]).wait()
        pltpu.make_async_copy(xtab_hbm.at[:, pl.ds(0, BLOCK)],
                              blkx_v.at[:, pl.ds(slot_off, BLOCK)],
                              sems.at[NUM_SLOTS + slot]).wait()

        colc = jnp.full((LANES,), slot_off, jnp.int32) + rc
        colx = jnp.full((LANES,), slot_off, jnp.int32) + rx
        acc = jnp.zeros((LANES,), jnp.float32)
        for k in range(EMBED_DIM // LANES):
            rows = lane_iota + k * LANES
            c = plsc.load_gather(blkc_v, [rows, colc])
            x = plsc.load_gather(blkx_v, [rows, colx])
            acc = acc + c * x
        total = plsc.cumsum(acc)
        plsc.store_scatter(out_v, [jnp.full((LANES,), j, jnp.int32)], total,
                           mask=last_mask)

        @pl.when(j + NUM_SLOTS < ROWS_PER_WORKER)
        def _():
            _fetch_row(ctab_hbm, blkc_v, idxc_v, j + NUM_SLOTS, sems, 0)
            _fetch_row(xtab_hbm, blkx_v, idxx_v, j + NUM_SLOTS, sems, 1)

        return carry

    lax.fori_loop(0, ROWS_PER_WORKER, step, 0)

    def sig_body(chunk, carry):
        sl = pl.ds(chunk * LANES, LANES)
        out_v[sl] = 1.0 / (1.0 + jnp.exp(-out_v[sl]))
        return carry

    lax.fori_loop(0, ROWS_PER_WORKER // LANES, sig_body, 0)

    pltpu.sync_copy(out_v, out_hbm.at[pl.ds(base, ROWS_PER_WORKER)])


def kernel(center_words, context_words, center_table, context_table):
    mesh = plsc.VectorSubcoreMesh(core_axis_name="c", subcore_axis_name="s")
    run = pl.kernel(
        _sc_body,
        out_type=jax.ShapeDtypeStruct((BATCH,), jnp.float32),
        mesh=mesh,
        scratch_types=[
            pltpu.VMEM((IDX_PAD,), jnp.int32),
            pltpu.VMEM((IDX_PAD,), jnp.int32),
            pltpu.VMEM((EMBED_DIM, NUM_SLOTS * BLOCK), jnp.float32),
            pltpu.VMEM((EMBED_DIM, NUM_SLOTS * BLOCK), jnp.float32),
            pltpu.VMEM((ROWS_PER_WORKER,), jnp.float32),
            pltpu.SemaphoreType.DMA((2 * NUM_SLOTS,)),
        ],
        compiler_params=pltpu.CompilerParams(
            needs_layout_passes=False, use_tc_tiling_on_sc=True),
    )
    return run(center_words.astype(jnp.int32), context_words.astype(jnp.int32),
               center_table.T, context_table.T)
